# baseline (speedup 1.0000x reference)
import functools

import jax
import jax.numpy as jnp
from jax import lax
from jax.experimental import pallas as pl
from jax.experimental.pallas import tpu as pltpu
from jax.experimental.pallas import tpu_sc as plsc

_NC = 2
_NS = 16
_NW = _NC * _NS
_L = 16


@functools.lru_cache(maxsize=None)
def _build_sc_call(B, H, W, D):
    HW = H * W
    ROWS = 2 * D
    RPW = ROWS // _NW
    SLICE = RPW * HW
    assert ROWS % _NW == 0 and W % _L == 0 and H % _L == 0

    mesh = plsc.VectorSubcoreMesh(core_axis_name="c", subcore_axis_name="s")

    @functools.partial(
        pl.kernel,
        out_type=jax.ShapeDtypeStruct((B * ROWS * HW,), jnp.float32),
        mesh=mesh,
        scratch_types=[
            pltpu.VMEM((H * D,), jnp.float32),
            pltpu.VMEM((SLICE,), jnp.float32),
            pltpu.SemaphoreType.DMA,
        ],
        compiler_params=pltpu.CompilerParams(needs_layout_passes=False),
    )
    def sc_call(col_hbm, row_hbm, out_hbm, emb_v, pos_v, sem):
        cid = lax.axis_index("c")
        sid = lax.axis_index("s")
        wid = sid * _NC + cid
        d0 = pl.multiple_of(wid * RPW, RPW)
        is_x = wid < _NW // 2

        @pl.when(is_x)
        def _():
            pltpu.sync_copy(col_hbm, emb_v)

        @pl.when(jnp.logical_not(is_x))
        def _():
            pltpu.sync_copy(row_hbm, emb_v)

        iota = lax.iota(jnp.int32, _L)

        @pl.when(is_x)
        def _():
            for j in range(RPW):
                cj = jnp.full((_L,), d0 + j, jnp.int32)
                vlo = plsc.load_gather(emb_v, [iota * D + cj])
                vhi = plsc.load_gather(emb_v, [(iota + _L) * D + cj])
                for h in range(H):
                    pos_v[pl.ds(j * HW + h * W, _L)] = vlo
                    pos_v[pl.ds(j * HW + h * W + _L, _L)] = vhi

        @pl.when(jnp.logical_not(is_x))
        def _():
            for j in range(RPW):
                cj = jnp.full((_L,), d0 - D + j, jnp.int32)
                for h in range(H):
                    v = plsc.load_gather(emb_v, [cj + h * D])
                    pos_v[pl.ds(j * HW + h * W, _L)] = v
                    pos_v[pl.ds(j * HW + h * W + _L, _L)] = v

        copies = []
        for b in range(B):
            off = pl.multiple_of((b * ROWS + d0) * HW, SLICE)
            copies.append(
                pltpu.make_async_copy(pos_v, out_hbm.at[pl.ds(off, SLICE)], sem)
            )
        for cp in copies:
            cp.start()
        for cp in copies:
            cp.wait()

    return sc_call


def kernel(pixel_values, row_embeddings, column_embeddings):
    B = pixel_values.shape[0]
    H, W = pixel_values.shape[-2], pixel_values.shape[-1]
    D = row_embeddings.shape[-1]
    col = column_embeddings[:W].reshape(-1)
    row = row_embeddings[:H].reshape(-1)
    out = _build_sc_call(B, H, W, D)(col, row)
    return out.reshape(B, 2 * D, H, W)


# speedup vs baseline: 2.5198x; 2.5198x over previous
import functools

import jax
import jax.numpy as jnp
from jax import lax
from jax.experimental import pallas as pl
from jax.experimental.pallas import tpu as pltpu
from jax.experimental.pallas import tpu_sc as plsc

_NC = 2
_NS = 16
_NW = _NC * _NS
_L = 16


@functools.lru_cache(maxsize=None)
def _build_sc_call(B, H, W, D):
    D2 = 2 * D
    PLANE = W * D2
    assert H == _NW and D % _L == 0

    mesh = plsc.VectorSubcoreMesh(core_axis_name="c", subcore_axis_name="s")

    @functools.partial(
        pl.kernel,
        out_type=jax.ShapeDtypeStruct((B * H * PLANE,), jnp.float32),
        mesh=mesh,
        scratch_types=[
            pltpu.VMEM((W * D,), jnp.float32),
            pltpu.VMEM((D,), jnp.float32),
            pltpu.VMEM((PLANE,), jnp.float32),
            pltpu.SemaphoreType.DMA,
        ],
        compiler_params=pltpu.CompilerParams(needs_layout_passes=False),
    )
    def sc_call(col_hbm, row_hbm, out_hbm, col_v, row_v, plane_v, sem):
        cid = lax.axis_index("c")
        sid = lax.axis_index("s")
        h = sid * _NC + cid
        pltpu.sync_copy(col_hbm, col_v)
        roff = pl.multiple_of(h * D, D)
        pltpu.sync_copy(row_hbm.at[pl.ds(roff, D)], row_v)

        rvecs = [row_v[pl.ds(t * _L, _L)] for t in range(D // _L)]
        for w in range(W):
            for t in range(D // _L):
                plane_v[pl.ds(w * D2 + t * _L, _L)] = col_v[
                    pl.ds(w * D + t * _L, _L)
                ]
                plane_v[pl.ds(w * D2 + D + t * _L, _L)] = rvecs[t]

        copies = []
        for b in range(B):
            off = pl.multiple_of((b * H + h) * PLANE, PLANE)
            copies.append(
                pltpu.make_async_copy(plane_v, out_hbm.at[pl.ds(off, PLANE)], sem)
            )
        for cp in copies:
            cp.start()
        for cp in copies:
            cp.wait()

    return sc_call


def kernel(pixel_values, row_embeddings, column_embeddings):
    B = pixel_values.shape[0]
    H, W = pixel_values.shape[-2], pixel_values.shape[-1]
    D = row_embeddings.shape[-1]
    col = column_embeddings[:W].reshape(-1)
    row = row_embeddings[:H].reshape(-1)
    out = _build_sc_call(B, H, W, D)(col, row)
    out4 = out.reshape(B, H, W, 2 * D)
    return jnp.transpose(out4, (0, 3, 1, 2))


# speedup vs baseline: 5.0155x; 1.9904x over previous
import functools

import jax
import jax.numpy as jnp
from jax import lax
from jax.experimental import pallas as pl
from jax.experimental.pallas import tpu as pltpu
from jax.experimental.pallas import tpu_sc as plsc

_NC = 2
_NS = 16
_NW = _NC * _NS
_L = 16


@functools.lru_cache(maxsize=None)
def _build_sc_call(B, H, W, D):
    D2 = 2 * D
    PLANE = W * D2
    assert H == _NW and D % _L == 0

    mesh = plsc.VectorSubcoreMesh(core_axis_name="c", subcore_axis_name="s")

    @functools.partial(
        pl.kernel,
        out_type=jax.ShapeDtypeStruct((B * H * PLANE,), jnp.float32),
        mesh=mesh,
        scratch_types=[
            pltpu.VMEM((W * D,), jnp.float32),
            pltpu.VMEM((D,), jnp.float32),
            pltpu.VMEM((PLANE,), jnp.float32),
            pltpu.SemaphoreType.DMA,
        ],
        compiler_params=pltpu.CompilerParams(needs_layout_passes=False),
    )
    def sc_call(col_hbm, row_hbm, out_hbm, col_v, row_v, plane_v, sem):
        cid = lax.axis_index("c")
        sid = lax.axis_index("s")
        h = sid * _NC + cid
        pltpu.sync_copy(col_hbm, col_v)
        roff = pl.multiple_of(h * D, D)
        pltpu.sync_copy(row_hbm.at[pl.ds(roff, D)], row_v)

        TW, TD = 8, 128
        WT, DT = W // TW, D2 // TD
        DHALF = D // TD
        rvecs = [row_v[pl.ds(t * _L, _L)] for t in range(D // _L)]
        for wt in range(WT):
            for wi in range(TW):
                w = wt * TW + wi
                for dt in range(DT):
                    dst = (wt * DT + dt) * TW * TD + wi * TD
                    if dt < DHALF:
                        for t in range(TD // _L):
                            plane_v[pl.ds(dst + t * _L, _L)] = col_v[
                                pl.ds(w * D + dt * TD + t * _L, _L)
                            ]
                    else:
                        for t in range(TD // _L):
                            plane_v[pl.ds(dst + t * _L, _L)] = rvecs[
                                (dt - DHALF) * (TD // _L) + t
                            ]

        copies = []
        for b in range(B):
            off = pl.multiple_of((b * H + h) * PLANE, PLANE)
            copies.append(
                pltpu.make_async_copy(plane_v, out_hbm.at[pl.ds(off, PLANE)], sem)
            )
        for cp in copies:
            cp.start()
        for cp in copies:
            cp.wait()

    return sc_call


def kernel(pixel_values, row_embeddings, column_embeddings):
    B = pixel_values.shape[0]
    H, W = pixel_values.shape[-2], pixel_values.shape[-1]
    D = row_embeddings.shape[-1]
    col = column_embeddings[:W].reshape(-1)
    row = row_embeddings[:H].reshape(-1)
    out = _build_sc_call(B, H, W, D)(col, row)
    out6 = out.reshape(B, H, W // 8, (2 * D) // 128, 8, 128)
    return jnp.transpose(out6, (0, 3, 5, 1, 2, 4)).reshape(B, 2 * D, H, W)
